# baseline (speedup 1.0000x reference)
import functools

import jax
import jax.numpy as jnp
from jax import lax
from jax.experimental import pallas as pl
from jax.experimental.pallas import tpu as pltpu
from jax.experimental.pallas import tpu_sc as plsc

N = 10000
E = 160000
F_IN = 256
HID = 256
OUT = 128

NC = 2
NS = 16
CH = 128
EP = 163840
NPAD = 16
CNTP = 10112

_PREC = jax.lax.Precision.DEFAULT


_RB = 1000


def _silu(v):
    return v * (1.0 / (1.0 + jnp.exp(-v)))


def _ln(v, g, b, eps=1e-5):
    mu = jnp.mean(v, axis=-1, keepdims=True)
    var = jnp.mean((v - mu) * (v - mu), axis=-1, keepdims=True)
    return (v - mu) / jnp.sqrt(var + eps) * g + b


def _dense1_body(x_ref, wl1_ref, wr1_ref, wlin_ref, blin_ref,
                 y1_ref, xr_ref, res_ref):
    xa = x_ref[...]
    y1 = jnp.dot(xa, wl1_ref[...], preferred_element_type=jnp.float32,
                 precision=_PREC)
    y1_ref[0] = y1[:, :HID // 2]
    y1_ref[1] = y1[:, HID // 2:]
    xr_ref[...] = jnp.dot(xa, wr1_ref[...], preferred_element_type=jnp.float32,
                          precision=_PREC)
    res_ref[...] = jnp.dot(xa, wlin_ref[...], preferred_element_type=jnp.float32,
                           precision=_PREC) + blin_ref[...]


def _dense1(x, W_l1, W_r1, W_lin, b_lin2d):
    grid = (N // _RB,)
    return pl.pallas_call(
        _dense1_body,
        grid=grid,
        in_specs=[
            pl.BlockSpec((_RB, F_IN), lambda i: (i, 0)),
            pl.BlockSpec((F_IN, HID), lambda i: (0, 0)),
            pl.BlockSpec((F_IN, HID), lambda i: (0, 0)),
            pl.BlockSpec((F_IN, HID), lambda i: (0, 0)),
            pl.BlockSpec((1, HID), lambda i: (0, 0)),
        ],
        out_specs=[
            pl.BlockSpec((NC, _RB, HID // NC), lambda i: (0, i, 0)),
            pl.BlockSpec((_RB, HID), lambda i: (i, 0)),
            pl.BlockSpec((_RB, HID), lambda i: (i, 0)),
        ],
        out_shape=[
            jax.ShapeDtypeStruct((NC, N, HID // NC), jnp.float32),
            jax.ShapeDtypeStruct((N, HID), jnp.float32),
            jax.ShapeDtypeStruct((N, HID), jnp.float32),
        ],
    )(x, W_l1, W_r1, W_lin, b_lin2d)


def _dense2_body(agg_ref, cnt_ref, xr_ref, res_ref, bl1_ref, g1_ref, be1_ref,
                 g2_ref, be2_ref, wl2_ref, wr2_ref, y2_ref, hr_ref):
    cm = jnp.maximum(cnt_ref[...], 1.0)
    mean = jnp.concatenate([agg_ref[0], agg_ref[1]], axis=-1) / cm
    h = mean + bl1_ref[...] + xr_ref[...]
    h = _silu(h)
    h = _ln(h, g1_ref[...], be1_ref[...])
    h = res_ref[...] + h
    h = _silu(h)
    h = _ln(h, g2_ref[...], be2_ref[...])
    y2 = jnp.dot(h, wl2_ref[...], preferred_element_type=jnp.float32,
                 precision=_PREC)
    y2_ref[0] = y2
    y2_ref[1] = y2
    hr_ref[...] = jnp.dot(h, wr2_ref[...], preferred_element_type=jnp.float32,
                          precision=_PREC)


def _dense2(agg1, cnt, xr, res, b_l1, g1, be1, g2, be2, W_l2, W_r2):
    grid = (N // _RB,)
    vec = lambda: pl.BlockSpec((1, HID), lambda i: (0, 0))
    return pl.pallas_call(
        _dense2_body,
        grid=grid,
        in_specs=[
            pl.BlockSpec((NC, _RB, HID // NC), lambda i: (0, i, 0)),
            pl.BlockSpec((_RB, 1), lambda i: (i, 0)),
            pl.BlockSpec((_RB, HID), lambda i: (i, 0)),
            pl.BlockSpec((_RB, HID), lambda i: (i, 0)),
            vec(), vec(), vec(), vec(), vec(),
            pl.BlockSpec((HID, OUT), lambda i: (0, 0)),
            pl.BlockSpec((HID, OUT), lambda i: (0, 0)),
        ],
        out_specs=[
            pl.BlockSpec((NC, _RB, OUT), lambda i: (0, i, 0)),
            pl.BlockSpec((_RB, OUT), lambda i: (i, 0)),
        ],
        out_shape=[
            jax.ShapeDtypeStruct((NC, N, OUT), jnp.float32),
            jax.ShapeDtypeStruct((N, OUT), jnp.float32),
        ],
    )(agg1, cnt, xr, res, b_l1, g1, be1, g2, be2, W_l2, W_r2)


def _dense3_body(agg_ref, cnt_ref, hr_ref, bl2_ref, go_ref, beo_ref, out_ref):
    cm = jnp.maximum(cnt_ref[...], 1.0)
    mean = (agg_ref[0] + agg_ref[1]) / cm
    h = mean + bl2_ref[...] + hr_ref[...]
    out_ref[...] = _ln(h, go_ref[...], beo_ref[...])


def _dense3(agg2, cnt, hr, b_l2, g_out, be_out):
    grid = (N // _RB,)
    vec = lambda: pl.BlockSpec((1, OUT), lambda i: (0, 0))
    return pl.pallas_call(
        _dense3_body,
        grid=grid,
        in_specs=[
            pl.BlockSpec((NC, _RB, OUT), lambda i: (0, i, 0)),
            pl.BlockSpec((_RB, 1), lambda i: (i, 0)),
            pl.BlockSpec((_RB, OUT), lambda i: (i, 0)),
            vec(), vec(), vec(),
        ],
        out_specs=pl.BlockSpec((_RB, OUT), lambda i: (i, 0)),
        out_shape=jax.ShapeDtypeStruct((N, OUT), jnp.float32),
    )(agg2, cnt, hr, b_l2, g_out, be_out)


def _make_seg(d, with_cnt, split):
    per_core = (EP // CH) // (1 if split == "feat" else NC)
    base = per_core // NS
    pblk = 40
    nph = base // pblk
    assert base == nph * pblk
    rq = (N // NS) // 8 * 8
    tail = N - rq * NS

    mesh = plsc.VectorSubcoreMesh(core_axis_name="c", subcore_axis_name="s",
                                  num_cores=NC, num_subcores=NS)

    out_type = [jax.ShapeDtypeStruct((NC, N, d), jnp.float32)]
    scratch = [
        pltpu.VMEM((pblk, CH), jnp.int32),
        pltpu.VMEM((pblk, CH), jnp.int32),
        pltpu.VMEM((CH, d), jnp.float32),
        pltpu.VMEM((CH, d), jnp.float32),
        pltpu.VMEM_SHARED((N + NPAD, d), jnp.float32),
        pltpu.SemaphoreType.DMA,
        pltpu.SemaphoreType.DMA,
        pltpu.SemaphoreType.DMA,
        pltpu.SemaphoreType.DMA,
        pltpu.SemaphoreType.DMA,
    ]
    if with_cnt:
        out_type.append(jax.ShapeDtypeStruct((NC, CNTP), jnp.float32))
        scratch += [
            pltpu.VMEM((5 * CH,), jnp.float32),
            pltpu.VMEM_SHARED((CNTP,), jnp.float32),
        ]
    cq = [5 * CH] * (NS - 1) + [CNTP - 5 * CH * (NS - 1)]
    assert cq[-1] > 0 and cq[-1] % CH == 0

    def body(tbl, src3, dst3, zer, out, *rest):
        if with_cnt:
            (cnt_out, sidx, didx, rbufa, rbufb, acc, sga, sgb, ssa, ssb, semc,
             ones, cacc) = rest
        else:
            sidx, didx, rbufa, rbufb, acc, sga, sgb, ssa, ssb, semc = rest
        c = lax.axis_index("c")
        s = lax.axis_index("s")
        start = s * base
        if split == "edge":
            start = c * per_core + start

        pltpu.sync_copy(zer.at[pl.ds(s * rq, rq)], acc.at[pl.ds(s * rq, rq)])
        @pl.when(s == NS - 1)
        def _():
            pltpu.sync_copy(zer.at[pl.ds(NS * rq, tail)],
                            acc.at[pl.ds(NS * rq, tail)])
        if with_cnt:
            def fill(i, _):
                ones[pl.ds(i * 16, 16)] = jnp.where(
                    i < CH // 16, 1.0, 0.0) * jnp.ones((16,), jnp.float32)
                return 0
            lax.fori_loop(0, 5 * CH // 16, fill, 0, unroll=False)
            @pl.when(s < NS - 1)
            def _():
                pltpu.sync_copy(ones.at[pl.ds(CH, 4 * CH)],
                                cacc.at[pl.ds(s * 5 * CH, 4 * CH)])
                pltpu.sync_copy(ones.at[pl.ds(CH, CH)],
                                cacc.at[pl.ds(s * 5 * CH + 4 * CH, CH)])
            @pl.when(s == NS - 1)
            def _():
                pltpu.sync_copy(ones.at[pl.ds(CH, 4 * CH)],
                                cacc.at[pl.ds((NS - 1) * 5 * CH, 4 * CH)])

        plsc.subcore_barrier()

        for ph in range(nph):
            pltpu.sync_copy(src3.at[c, pl.ds(start + ph * pblk, pblk)], sidx)
            pltpu.sync_copy(dst3.at[0, pl.ds(start + ph * pblk, pblk)], didx)
            pltpu.async_copy(tbl.at[sidx.at[0]], rbufa, sga)

            def chunk(j, _):
                def work(cur, oth, sg_c, ss_c, sg_o, ss_o):
                    @pl.when(j >= 1)
                    def _():
                        pltpu.make_async_copy(
                            oth, acc.at[didx.at[0]], ss_o).wait()
                    @pl.when(j < pblk - 1)
                    def _():
                        pltpu.async_copy(tbl.at[sidx.at[j + 1]], oth, sg_o)
                    pltpu.make_async_copy(tbl.at[sidx.at[j]], cur, sg_c).wait()
                    pltpu.async_copy(cur, acc.at[didx.at[j]], ss_c, add=True)
                @pl.when(lax.rem(j, 2) == 0)
                def _():
                    work(rbufa, rbufb, sga, ssa, sgb, ssb)
                @pl.when(lax.rem(j, 2) == 1)
                def _():
                    work(rbufb, rbufa, sgb, ssb, sga, ssa)
                if with_cnt:
                    @pl.when(c == ph % NC)
                    def _():
                        @pl.when(j >= 1)
                        def _():
                            pltpu.make_async_copy(
                                ones.at[pl.ds(0, CH)], cacc.at[didx.at[0]],
                                semc).wait()
                        pltpu.async_copy(ones.at[pl.ds(0, CH)],
                                         cacc.at[didx.at[j]], semc, add=True)
                return 0

            lax.fori_loop(0, pblk, chunk, 0, unroll=False)

            last = rbufb if (pblk - 1) % 2 else rbufa
            lsem = ssb if (pblk - 1) % 2 else ssa
            pltpu.make_async_copy(last, acc.at[didx.at[0]], lsem).wait()
            if with_cnt:
                @pl.when(c == ph % NC)
                def _():
                    pltpu.make_async_copy(ones.at[pl.ds(0, CH)],
                                          cacc.at[didx.at[0]], semc).wait()

        plsc.subcore_barrier()

        pltpu.sync_copy(acc.at[pl.ds(s * rq, rq)],
                        out.at[c, pl.ds(s * rq, rq)])
        @pl.when(s == NS - 1)
        def _():
            pltpu.sync_copy(acc.at[pl.ds(NS * rq, tail)],
                            out.at[c, pl.ds(NS * rq, tail)])
        if with_cnt:
            @pl.when(s < NS - 1)
            def _():
                pltpu.sync_copy(cacc.at[pl.ds(s * 5 * CH, 5 * CH)],
                                cnt_out.at[c, pl.ds(s * 5 * CH, 5 * CH)])
            @pl.when(s == NS - 1)
            def _():
                pltpu.sync_copy(cacc.at[pl.ds((NS - 1) * 5 * CH, 4 * CH)],
                                cnt_out.at[c, pl.ds((NS - 1) * 5 * CH, 4 * CH)])

    return pl.kernel(body, out_type=out_type, mesh=mesh, scratch_types=scratch)


def kernel(x, edge_index, W_lin, b_lin, W_l1, b_l1, W_r1, W_l2, b_l2, W_r2,
           g1, be1, g2, be2, g_out, be_out):
    src = edge_index[0].astype(jnp.int32)
    dst = edge_index[1].astype(jnp.int32)
    nch_all = EP // CH
    pad_per = CH - E // nch_all
    dsrc = jnp.zeros((nch_all, pad_per), jnp.int32)
    ddst = N + (jnp.arange(nch_all * pad_per, dtype=jnp.int32)
                % NPAD).reshape(nch_all, pad_per)
    src = jnp.concatenate([src.reshape(nch_all, CH - pad_per), dsrc], axis=1)
    dst = jnp.concatenate([dst.reshape(nch_all, CH - pad_per), ddst], axis=1)
    src = src.reshape(EP)
    dst = dst.reshape(EP)
    src3 = jnp.stack([src, src + N]).reshape(NC, EP // CH, CH)
    dst3 = dst.reshape(1, EP // CH, CH)

    zers = jnp.zeros((N + NPAD, HID // NC), jnp.float32)

    y1t, xr, res = _dense1(x, W_l1, W_r1, W_lin, b_lin.reshape(1, HID))
    agg1, cnt2 = _make_seg(HID // NC, True, "feat")(
        y1t.reshape(NC * N, HID // NC), src3, dst3, zers)
    cnt = (cnt2[0] + cnt2[1])[:N].reshape(N, 1)
    y2, hr = _dense2(agg1, cnt, xr, res,
                     b_l1.reshape(1, HID), g1.reshape(1, HID),
                     be1.reshape(1, HID), g2.reshape(1, HID),
                     be2.reshape(1, HID), W_l2, W_r2)
    (agg2,) = _make_seg(OUT, False, "edge")(
        y2.reshape(NC * N, OUT), src3, dst3, zers)
    out = _dense3(agg2, cnt, hr, b_l2.reshape(1, OUT),
                  g_out.reshape(1, OUT), be_out.reshape(1, OUT))
    return out


# speedup vs baseline: 1.0108x; 1.0108x over previous
import functools

import jax
import jax.numpy as jnp
from jax import lax
from jax.experimental import pallas as pl
from jax.experimental.pallas import tpu as pltpu
from jax.experimental.pallas import tpu_sc as plsc

N = 10000
E = 160000
F_IN = 256
HID = 256
OUT = 128

NC = 2
NS = 16
CH = 128
EP = 163840
NPAD = 16
CNTP = 10112

_PREC = jax.lax.Precision.DEFAULT


_RB = 1000


def _silu(v):
    return v * (1.0 / (1.0 + jnp.exp(-v)))


def _ln(v, g, b, eps=1e-5):
    mu = jnp.mean(v, axis=-1, keepdims=True)
    var = jnp.mean((v - mu) * (v - mu), axis=-1, keepdims=True)
    return (v - mu) / jnp.sqrt(var + eps) * g + b


def _dense1_body(x_ref, wl1_ref, wr1_ref, wlin_ref, blin_ref,
                 y1_ref, xr_ref, res_ref):
    xa = x_ref[...]
    y1 = jnp.dot(xa, wl1_ref[...], preferred_element_type=jnp.float32,
                 precision=_PREC)
    y1_ref[0] = y1[:, :HID // 2]
    y1_ref[1] = y1[:, HID // 2:]
    xr_ref[...] = jnp.dot(xa, wr1_ref[...], preferred_element_type=jnp.float32,
                          precision=_PREC)
    res_ref[...] = jnp.dot(xa, wlin_ref[...], preferred_element_type=jnp.float32,
                           precision=_PREC) + blin_ref[...]


def _dense1(x, W_l1, W_r1, W_lin, b_lin2d):
    grid = (N // _RB,)
    return pl.pallas_call(
        _dense1_body,
        grid=grid,
        in_specs=[
            pl.BlockSpec((_RB, F_IN), lambda i: (i, 0)),
            pl.BlockSpec((F_IN, HID), lambda i: (0, 0)),
            pl.BlockSpec((F_IN, HID), lambda i: (0, 0)),
            pl.BlockSpec((F_IN, HID), lambda i: (0, 0)),
            pl.BlockSpec((1, HID), lambda i: (0, 0)),
        ],
        out_specs=[
            pl.BlockSpec((NC, _RB, HID // NC), lambda i: (0, i, 0)),
            pl.BlockSpec((_RB, HID), lambda i: (i, 0)),
            pl.BlockSpec((_RB, HID), lambda i: (i, 0)),
        ],
        out_shape=[
            jax.ShapeDtypeStruct((NC, N, HID // NC), jnp.float32),
            jax.ShapeDtypeStruct((N, HID), jnp.float32),
            jax.ShapeDtypeStruct((N, HID), jnp.float32),
        ],
    )(x, W_l1, W_r1, W_lin, b_lin2d)


def _dense2_body(agg_ref, cnt_ref, xr_ref, res_ref, bl1_ref, g1_ref, be1_ref,
                 g2_ref, be2_ref, wl2_ref, wr2_ref, y2_ref, hr_ref):
    cm = jnp.maximum(cnt_ref[...], 1.0)
    mean = jnp.concatenate([agg_ref[0], agg_ref[1]], axis=-1) / cm
    h = mean + bl1_ref[...] + xr_ref[...]
    h = _silu(h)
    h = _ln(h, g1_ref[...], be1_ref[...])
    h = res_ref[...] + h
    h = _silu(h)
    h = _ln(h, g2_ref[...], be2_ref[...])
    y2 = jnp.dot(h, wl2_ref[...], preferred_element_type=jnp.float32,
                 precision=_PREC)
    y2_ref[0] = y2
    y2_ref[1] = y2
    hr_ref[...] = jnp.dot(h, wr2_ref[...], preferred_element_type=jnp.float32,
                          precision=_PREC)


def _dense2(agg1, cnt, xr, res, b_l1, g1, be1, g2, be2, W_l2, W_r2):
    grid = (N // _RB,)
    vec = lambda: pl.BlockSpec((1, HID), lambda i: (0, 0))
    return pl.pallas_call(
        _dense2_body,
        grid=grid,
        in_specs=[
            pl.BlockSpec((NC, _RB, HID // NC), lambda i: (0, i, 0)),
            pl.BlockSpec((_RB, 1), lambda i: (i, 0)),
            pl.BlockSpec((_RB, HID), lambda i: (i, 0)),
            pl.BlockSpec((_RB, HID), lambda i: (i, 0)),
            vec(), vec(), vec(), vec(), vec(),
            pl.BlockSpec((HID, OUT), lambda i: (0, 0)),
            pl.BlockSpec((HID, OUT), lambda i: (0, 0)),
        ],
        out_specs=[
            pl.BlockSpec((NC, _RB, OUT), lambda i: (0, i, 0)),
            pl.BlockSpec((_RB, OUT), lambda i: (i, 0)),
        ],
        out_shape=[
            jax.ShapeDtypeStruct((NC, N, OUT), jnp.float32),
            jax.ShapeDtypeStruct((N, OUT), jnp.float32),
        ],
    )(agg1, cnt, xr, res, b_l1, g1, be1, g2, be2, W_l2, W_r2)


def _dense3_body(agg_ref, cnt_ref, hr_ref, bl2_ref, go_ref, beo_ref, out_ref):
    cm = jnp.maximum(cnt_ref[...], 1.0)
    mean = (agg_ref[0] + agg_ref[1]) / cm
    h = mean + bl2_ref[...] + hr_ref[...]
    out_ref[...] = _ln(h, go_ref[...], beo_ref[...])


def _dense3(agg2, cnt, hr, b_l2, g_out, be_out):
    grid = (N // _RB,)
    vec = lambda: pl.BlockSpec((1, OUT), lambda i: (0, 0))
    return pl.pallas_call(
        _dense3_body,
        grid=grid,
        in_specs=[
            pl.BlockSpec((NC, _RB, OUT), lambda i: (0, i, 0)),
            pl.BlockSpec((_RB, 1), lambda i: (i, 0)),
            pl.BlockSpec((_RB, OUT), lambda i: (i, 0)),
            vec(), vec(), vec(),
        ],
        out_specs=pl.BlockSpec((_RB, OUT), lambda i: (i, 0)),
        out_shape=jax.ShapeDtypeStruct((N, OUT), jnp.float32),
    )(agg2, cnt, hr, b_l2, g_out, be_out)


def _make_seg(d, with_cnt, split):
    per_core = (EP // CH) // (1 if split == "feat" else NC)
    base = per_core // NS
    pblk = 40
    nph = base // pblk
    assert base == nph * pblk
    rq = (N // NS) // 8 * 8
    tail = N - rq * NS

    mesh = plsc.VectorSubcoreMesh(core_axis_name="c", subcore_axis_name="s",
                                  num_cores=NC, num_subcores=NS)

    out_type = [jax.ShapeDtypeStruct((NC, N, d), jnp.float32)]
    scratch = [
        pltpu.VMEM((pblk, CH), jnp.int32),
        pltpu.VMEM((pblk, CH), jnp.int32),
        pltpu.VMEM((CH, d), jnp.float32),
        pltpu.VMEM((CH, d), jnp.float32),
        pltpu.VMEM_SHARED((N + NPAD, d), jnp.float32),
        pltpu.SemaphoreType.DMA,
        pltpu.SemaphoreType.DMA,
        pltpu.SemaphoreType.DMA,
        pltpu.SemaphoreType.DMA,
        pltpu.SemaphoreType.DMA,
    ]
    if with_cnt:
        out_type.append(jax.ShapeDtypeStruct((NC, CNTP), jnp.float32))
        scratch += [
            pltpu.VMEM((5 * CH,), jnp.float32),
            pltpu.VMEM_SHARED((CNTP,), jnp.float32),
        ]
    cq = [5 * CH] * (NS - 1) + [CNTP - 5 * CH * (NS - 1)]
    assert cq[-1] > 0 and cq[-1] % CH == 0

    def body(tbl, src3, dst3, zer, out, *rest):
        if with_cnt:
            (cnt_out, sidx, didx, rbufa, rbufb, acc, sga, sgb, ssa, ssb, semc,
             ones, cacc) = rest
        else:
            sidx, didx, rbufa, rbufb, acc, sga, sgb, ssa, ssb, semc = rest
        c = lax.axis_index("c")
        s = lax.axis_index("s")
        start = s * base
        if split == "edge":
            start = c * per_core + start

        pltpu.sync_copy(zer.at[pl.ds(s * rq, rq)], acc.at[pl.ds(s * rq, rq)])
        @pl.when(s == NS - 1)
        def _():
            pltpu.sync_copy(zer.at[pl.ds(NS * rq, tail)],
                            acc.at[pl.ds(NS * rq, tail)])
        if with_cnt:
            def fill(i, _):
                ones[pl.ds(i * 16, 16)] = jnp.where(
                    i < CH // 16, 1.0, 0.0) * jnp.ones((16,), jnp.float32)
                return 0
            lax.fori_loop(0, 5 * CH // 16, fill, 0, unroll=False)
            @pl.when(s < NS - 1)
            def _():
                pltpu.sync_copy(ones.at[pl.ds(CH, 4 * CH)],
                                cacc.at[pl.ds(s * 5 * CH, 4 * CH)])
                pltpu.sync_copy(ones.at[pl.ds(CH, CH)],
                                cacc.at[pl.ds(s * 5 * CH + 4 * CH, CH)])
            @pl.when(s == NS - 1)
            def _():
                pltpu.sync_copy(ones.at[pl.ds(CH, 4 * CH)],
                                cacc.at[pl.ds((NS - 1) * 5 * CH, 4 * CH)])

        plsc.subcore_barrier()

        for ph in range(nph):
            pltpu.sync_copy(src3.at[c, pl.ds(start + ph * pblk, pblk)], sidx)
            pltpu.sync_copy(dst3.at[c, pl.ds(start + ph * pblk, pblk)], didx)
            pltpu.async_copy(tbl.at[sidx.at[0]], rbufa, sga)

            def chunk(j, _):
                def work(cur, oth, sg_c, ss_c, sg_o, ss_o):
                    @pl.when(j >= 1)
                    def _():
                        pltpu.make_async_copy(
                            oth, acc.at[didx.at[0]], ss_o).wait()
                    @pl.when(j < pblk - 1)
                    def _():
                        pltpu.async_copy(tbl.at[sidx.at[j + 1]], oth, sg_o)
                    pltpu.make_async_copy(tbl.at[sidx.at[j]], cur, sg_c).wait()
                    pltpu.async_copy(cur, acc.at[didx.at[j]], ss_c, add=True)
                @pl.when(lax.rem(j, 2) == 0)
                def _():
                    work(rbufa, rbufb, sga, ssa, sgb, ssb)
                @pl.when(lax.rem(j, 2) == 1)
                def _():
                    work(rbufb, rbufa, sgb, ssb, sga, ssa)
                if with_cnt:
                    @pl.when(c == ph % NC)
                    def _():
                        @pl.when(j >= 1)
                        def _():
                            pltpu.make_async_copy(
                                ones.at[pl.ds(0, CH)], cacc.at[didx.at[0]],
                                semc).wait()
                        pltpu.async_copy(ones.at[pl.ds(0, CH)],
                                         cacc.at[didx.at[j]], semc, add=True)
                return 0

            lax.fori_loop(0, pblk, chunk, 0, unroll=False)

            last = rbufb if (pblk - 1) % 2 else rbufa
            lsem = ssb if (pblk - 1) % 2 else ssa
            pltpu.make_async_copy(last, acc.at[didx.at[0]], lsem).wait()
            if with_cnt:
                @pl.when(c == ph % NC)
                def _():
                    pltpu.make_async_copy(ones.at[pl.ds(0, CH)],
                                          cacc.at[didx.at[0]], semc).wait()

        plsc.subcore_barrier()

        pltpu.sync_copy(acc.at[pl.ds(s * rq, rq)],
                        out.at[c, pl.ds(s * rq, rq)])
        @pl.when(s == NS - 1)
        def _():
            pltpu.sync_copy(acc.at[pl.ds(NS * rq, tail)],
                            out.at[c, pl.ds(NS * rq, tail)])
        if with_cnt:
            @pl.when(s < NS - 1)
            def _():
                pltpu.sync_copy(cacc.at[pl.ds(s * 5 * CH, 5 * CH)],
                                cnt_out.at[c, pl.ds(s * 5 * CH, 5 * CH)])
            @pl.when(s == NS - 1)
            def _():
                pltpu.sync_copy(cacc.at[pl.ds((NS - 1) * 5 * CH, 4 * CH)],
                                cnt_out.at[c, pl.ds((NS - 1) * 5 * CH, 4 * CH)])

    return pl.kernel(body, out_type=out_type, mesh=mesh, scratch_types=scratch)


def kernel(x, edge_index, W_lin, b_lin, W_l1, b_l1, W_r1, W_l2, b_l2, W_r2,
           g1, be1, g2, be2, g_out, be_out):
    src = edge_index[0].astype(jnp.int32)
    dst = edge_index[1].astype(jnp.int32)
    nch_all = EP // CH
    pad_per = CH - E // nch_all
    dsrc = jnp.zeros((nch_all, pad_per), jnp.int32)
    ddst = N + (jnp.arange(nch_all * pad_per, dtype=jnp.int32)
                % NPAD).reshape(nch_all, pad_per)
    src = jnp.concatenate([src.reshape(nch_all, CH - pad_per), dsrc], axis=1)
    dst = jnp.concatenate([dst.reshape(nch_all, CH - pad_per), ddst], axis=1)
    src = src.reshape(EP)
    dst = dst.reshape(EP)
    src3 = jnp.stack([src, src + N]).reshape(NC, EP // CH, CH)
    dst3 = jnp.tile(dst.reshape(1, EP // CH, CH), (NC, 1, 1))

    zers = jnp.zeros((N + NPAD, HID // NC), jnp.float32)

    y1t, xr, res = _dense1(x, W_l1, W_r1, W_lin, b_lin.reshape(1, HID))
    agg1, cnt2 = _make_seg(HID // NC, True, "feat")(
        y1t.reshape(NC * N, HID // NC), src3, dst3, zers)
    cnt = (cnt2[0] + cnt2[1])[:N].reshape(N, 1)
    y2, hr = _dense2(agg1, cnt, xr, res,
                     b_l1.reshape(1, HID), g1.reshape(1, HID),
                     be1.reshape(1, HID), g2.reshape(1, HID),
                     be2.reshape(1, HID), W_l2, W_r2)
    (agg2,) = _make_seg(OUT, False, "edge")(
        y2.reshape(NC * N, OUT), src3, dst3, zers)
    out = _dense3(agg2, cnt, hr, b_l2.reshape(1, OUT),
                  g_out.reshape(1, OUT), be_out.reshape(1, OUT))
    return out


# speedup vs baseline: 1.0255x; 1.0145x over previous
import functools

import jax
import jax.numpy as jnp
from jax import lax
from jax.experimental import pallas as pl
from jax.experimental.pallas import tpu as pltpu
from jax.experimental.pallas import tpu_sc as plsc

N = 10000
E = 160000
F_IN = 256
HID = 256
OUT = 128

NC = 2
NS = 16
CH = 128
EP = 163840
NPAD = 16
CNTP = 10112

_PREC = jax.lax.Precision.DEFAULT


_RB = 1000


def _silu(v):
    return v * (1.0 / (1.0 + jnp.exp(-v)))


def _ln(v, g, b, eps=1e-5):
    mu = jnp.mean(v, axis=-1, keepdims=True)
    var = jnp.mean((v - mu) * (v - mu), axis=-1, keepdims=True)
    return (v - mu) / jnp.sqrt(var + eps) * g + b


def _dense1_body(x_ref, wl1_ref, wr1_ref, wlin_ref, blin_ref,
                 y1_ref, xr_ref, res_ref):
    xa = x_ref[...]
    y1 = jnp.dot(xa, wl1_ref[...], preferred_element_type=jnp.float32,
                 precision=_PREC)
    y1_ref[0] = y1[:, :HID // 2]
    y1_ref[1] = y1[:, HID // 2:]
    xr_ref[...] = jnp.dot(xa, wr1_ref[...], preferred_element_type=jnp.float32,
                          precision=_PREC)
    res_ref[...] = jnp.dot(xa, wlin_ref[...], preferred_element_type=jnp.float32,
                           precision=_PREC) + blin_ref[...]


def _dense1(x, W_l1, W_r1, W_lin, b_lin2d):
    grid = (N // _RB,)
    return pl.pallas_call(
        _dense1_body,
        grid=grid,
        in_specs=[
            pl.BlockSpec((_RB, F_IN), lambda i: (i, 0)),
            pl.BlockSpec((F_IN, HID), lambda i: (0, 0)),
            pl.BlockSpec((F_IN, HID), lambda i: (0, 0)),
            pl.BlockSpec((F_IN, HID), lambda i: (0, 0)),
            pl.BlockSpec((1, HID), lambda i: (0, 0)),
        ],
        out_specs=[
            pl.BlockSpec((NC, _RB, HID // NC), lambda i: (0, i, 0)),
            pl.BlockSpec((_RB, HID), lambda i: (i, 0)),
            pl.BlockSpec((_RB, HID), lambda i: (i, 0)),
        ],
        out_shape=[
            jax.ShapeDtypeStruct((NC, N, HID // NC), jnp.float32),
            jax.ShapeDtypeStruct((N, HID), jnp.float32),
            jax.ShapeDtypeStruct((N, HID), jnp.float32),
        ],
    )(x, W_l1, W_r1, W_lin, b_lin2d)


def _dense2_body(agg_ref, cnt_ref, xr_ref, res_ref, bl1_ref, g1_ref, be1_ref,
                 g2_ref, be2_ref, wl2_ref, wr2_ref, y2_ref, hr_ref):
    cm = jnp.maximum(cnt_ref[...], 1.0)
    mean = jnp.concatenate([agg_ref[0], agg_ref[1]], axis=-1) / cm
    h = mean + bl1_ref[...] + xr_ref[...]
    h = _silu(h)
    h = _ln(h, g1_ref[...], be1_ref[...])
    h = res_ref[...] + h
    h = _silu(h)
    h = _ln(h, g2_ref[...], be2_ref[...])
    y2 = jnp.dot(h, wl2_ref[...], preferred_element_type=jnp.float32,
                 precision=_PREC)
    y2_ref[0] = y2
    y2_ref[1] = y2
    hr_ref[...] = jnp.dot(h, wr2_ref[...], preferred_element_type=jnp.float32,
                          precision=_PREC)


def _dense2(agg1, cnt, xr, res, b_l1, g1, be1, g2, be2, W_l2, W_r2):
    grid = (N // _RB,)
    vec = lambda: pl.BlockSpec((1, HID), lambda i: (0, 0))
    return pl.pallas_call(
        _dense2_body,
        grid=grid,
        in_specs=[
            pl.BlockSpec((NC, _RB, HID // NC), lambda i: (0, i, 0)),
            pl.BlockSpec((_RB, 1), lambda i: (i, 0)),
            pl.BlockSpec((_RB, HID), lambda i: (i, 0)),
            pl.BlockSpec((_RB, HID), lambda i: (i, 0)),
            vec(), vec(), vec(), vec(), vec(),
            pl.BlockSpec((HID, OUT), lambda i: (0, 0)),
            pl.BlockSpec((HID, OUT), lambda i: (0, 0)),
        ],
        out_specs=[
            pl.BlockSpec((NC, _RB, OUT), lambda i: (0, i, 0)),
            pl.BlockSpec((_RB, OUT), lambda i: (i, 0)),
        ],
        out_shape=[
            jax.ShapeDtypeStruct((NC, N, OUT), jnp.float32),
            jax.ShapeDtypeStruct((N, OUT), jnp.float32),
        ],
    )(agg1, cnt, xr, res, b_l1, g1, be1, g2, be2, W_l2, W_r2)


def _dense3_body(agg_ref, cnt_ref, hr_ref, bl2_ref, go_ref, beo_ref, out_ref):
    cm = jnp.maximum(cnt_ref[...], 1.0)
    mean = (agg_ref[0] + agg_ref[1]) / cm
    h = mean + bl2_ref[...] + hr_ref[...]
    out_ref[...] = _ln(h, go_ref[...], beo_ref[...])


def _dense3(agg2, cnt, hr, b_l2, g_out, be_out):
    grid = (N // _RB,)
    vec = lambda: pl.BlockSpec((1, OUT), lambda i: (0, 0))
    return pl.pallas_call(
        _dense3_body,
        grid=grid,
        in_specs=[
            pl.BlockSpec((NC, _RB, OUT), lambda i: (0, i, 0)),
            pl.BlockSpec((_RB, 1), lambda i: (i, 0)),
            pl.BlockSpec((_RB, OUT), lambda i: (i, 0)),
            vec(), vec(), vec(),
        ],
        out_specs=pl.BlockSpec((_RB, OUT), lambda i: (i, 0)),
        out_shape=jax.ShapeDtypeStruct((N, OUT), jnp.float32),
    )(agg2, cnt, hr, b_l2, g_out, be_out)


def _make_seg(d, with_cnt, split):
    per_core = (EP // CH) // (1 if split == "feat" else NC)
    base = per_core // NS
    pblk = 40
    nph = base // pblk
    assert base == nph * pblk
    rq = (N // NS) // 8 * 8
    tail = N - rq * NS

    mesh = plsc.VectorSubcoreMesh(core_axis_name="c", subcore_axis_name="s",
                                  num_cores=NC, num_subcores=NS)

    out_type = [jax.ShapeDtypeStruct((NC, N, d), jnp.float32)]
    scratch = [
        pltpu.VMEM((pblk, CH), jnp.int32),
        pltpu.VMEM((pblk, CH), jnp.int32),
        pltpu.VMEM((CH, d), jnp.float32),
        pltpu.VMEM((CH, d), jnp.float32),
        pltpu.VMEM_SHARED((N + NPAD, d), jnp.float32),
        pltpu.SemaphoreType.DMA,
        pltpu.SemaphoreType.DMA,
        pltpu.SemaphoreType.DMA,
        pltpu.SemaphoreType.DMA,
        pltpu.SemaphoreType.DMA,
    ]
    if with_cnt:
        out_type.append(jax.ShapeDtypeStruct((NC, CNTP), jnp.float32))
        scratch += [
            pltpu.VMEM((5 * CH,), jnp.float32),
            pltpu.VMEM_SHARED((CNTP,), jnp.float32),
        ]
    cq = [5 * CH] * (NS - 1) + [CNTP - 5 * CH * (NS - 1)]
    assert cq[-1] > 0 and cq[-1] % CH == 0

    def body(tbl, src3, dst3, zer, out, *rest):
        if with_cnt:
            (cnt_out, sidx, didx, rbufa, rbufb, acc, sga, sgb, ssa, ssb, semc,
             ones, cacc) = rest
        else:
            sidx, didx, rbufa, rbufb, acc, sga, sgb, ssa, ssb, semc = rest
        c = lax.axis_index("c")
        s = lax.axis_index("s")
        start = s * base


        pltpu.sync_copy(zer.at[pl.ds(s * rq, rq)], acc.at[pl.ds(s * rq, rq)])
        @pl.when(s == NS - 1)
        def _():
            pltpu.sync_copy(zer.at[pl.ds(NS * rq, tail)],
                            acc.at[pl.ds(NS * rq, tail)])
        if with_cnt:
            def fill(i, _):
                ones[pl.ds(i * 16, 16)] = jnp.where(
                    i < CH // 16, 1.0, 0.0) * jnp.ones((16,), jnp.float32)
                return 0
            lax.fori_loop(0, 5 * CH // 16, fill, 0, unroll=False)
            @pl.when(s < NS - 1)
            def _():
                pltpu.sync_copy(ones.at[pl.ds(CH, 4 * CH)],
                                cacc.at[pl.ds(s * 5 * CH, 4 * CH)])
                pltpu.sync_copy(ones.at[pl.ds(CH, CH)],
                                cacc.at[pl.ds(s * 5 * CH + 4 * CH, CH)])
            @pl.when(s == NS - 1)
            def _():
                pltpu.sync_copy(ones.at[pl.ds(CH, 4 * CH)],
                                cacc.at[pl.ds((NS - 1) * 5 * CH, 4 * CH)])

        plsc.subcore_barrier()

        for ph in range(nph):
            pltpu.sync_copy(src3.at[c, pl.ds(start + ph * pblk, pblk)], sidx)
            pltpu.sync_copy(dst3.at[c, pl.ds(start + ph * pblk, pblk)], didx)
            pltpu.async_copy(tbl.at[sidx.at[0]], rbufa, sga)

            def chunk(j, _):
                def work(cur, oth, sg_c, ss_c, sg_o, ss_o):
                    @pl.when(j >= 1)
                    def _():
                        pltpu.make_async_copy(
                            oth, acc.at[didx.at[0]], ss_o).wait()
                    @pl.when(j < pblk - 1)
                    def _():
                        pltpu.async_copy(tbl.at[sidx.at[j + 1]], oth, sg_o)
                    pltpu.make_async_copy(tbl.at[sidx.at[j]], cur, sg_c).wait()
                    pltpu.async_copy(cur, acc.at[didx.at[j]], ss_c, add=True)
                @pl.when(lax.rem(j, 2) == 0)
                def _():
                    work(rbufa, rbufb, sga, ssa, sgb, ssb)
                @pl.when(lax.rem(j, 2) == 1)
                def _():
                    work(rbufb, rbufa, sgb, ssb, sga, ssa)
                if with_cnt:
                    @pl.when(c == ph % NC)
                    def _():
                        @pl.when(j >= 1)
                        def _():
                            pltpu.make_async_copy(
                                ones.at[pl.ds(0, CH)], cacc.at[didx.at[0]],
                                semc).wait()
                        pltpu.async_copy(ones.at[pl.ds(0, CH)],
                                         cacc.at[didx.at[j]], semc, add=True)
                return 0

            lax.fori_loop(0, pblk, chunk, 0, unroll=False)

            last = rbufb if (pblk - 1) % 2 else rbufa
            lsem = ssb if (pblk - 1) % 2 else ssa
            pltpu.make_async_copy(last, acc.at[didx.at[0]], lsem).wait()
            if with_cnt:
                @pl.when(c == ph % NC)
                def _():
                    pltpu.make_async_copy(ones.at[pl.ds(0, CH)],
                                          cacc.at[didx.at[0]], semc).wait()

        plsc.subcore_barrier()

        pltpu.sync_copy(acc.at[pl.ds(s * rq, rq)],
                        out.at[c, pl.ds(s * rq, rq)])
        @pl.when(s == NS - 1)
        def _():
            pltpu.sync_copy(acc.at[pl.ds(NS * rq, tail)],
                            out.at[c, pl.ds(NS * rq, tail)])
        if with_cnt:
            @pl.when(s < NS - 1)
            def _():
                pltpu.sync_copy(cacc.at[pl.ds(s * 5 * CH, 5 * CH)],
                                cnt_out.at[c, pl.ds(s * 5 * CH, 5 * CH)])
            @pl.when(s == NS - 1)
            def _():
                pltpu.sync_copy(cacc.at[pl.ds((NS - 1) * 5 * CH, 4 * CH)],
                                cnt_out.at[c, pl.ds((NS - 1) * 5 * CH, 4 * CH)])

    return pl.kernel(body, out_type=out_type, mesh=mesh, scratch_types=scratch)


def kernel(x, edge_index, W_lin, b_lin, W_l1, b_l1, W_r1, W_l2, b_l2, W_r2,
           g1, be1, g2, be2, g_out, be_out):
    src = edge_index[0].astype(jnp.int32)
    dst = edge_index[1].astype(jnp.int32)
    nch_all = EP // CH
    pad_per = CH - E // nch_all
    dsrc = jnp.zeros((nch_all, pad_per), jnp.int32)
    ddst = N + (jnp.arange(nch_all * pad_per, dtype=jnp.int32)
                % NPAD).reshape(nch_all, pad_per)
    src = jnp.concatenate([src.reshape(nch_all, CH - pad_per), dsrc], axis=1)
    dst = jnp.concatenate([dst.reshape(nch_all, CH - pad_per), ddst], axis=1)
    src = src.reshape(EP)
    dst = dst.reshape(EP)
    src3 = jnp.stack([src, src + N]).reshape(NC, EP // CH, CH)
    dst3 = jnp.tile(dst.reshape(1, EP // CH, CH), (NC, 1, 1))
    src3e = (src.reshape(NC, EP // CH // NC, CH)
             + (jnp.arange(NC, dtype=jnp.int32) * N)[:, None, None])
    dst3e = dst.reshape(NC, EP // CH // NC, CH)

    zers = jnp.zeros((N + NPAD, HID // NC), jnp.float32)

    y1t, xr, res = _dense1(x, W_l1, W_r1, W_lin, b_lin.reshape(1, HID))
    agg1, cnt2 = _make_seg(HID // NC, True, "feat")(
        y1t.reshape(NC * N, HID // NC), src3, dst3, zers)
    cnt = (cnt2[0] + cnt2[1])[:N].reshape(N, 1)
    y2, hr = _dense2(agg1, cnt, xr, res,
                     b_l1.reshape(1, HID), g1.reshape(1, HID),
                     be1.reshape(1, HID), g2.reshape(1, HID),
                     be2.reshape(1, HID), W_l2, W_r2)
    (agg2,) = _make_seg(OUT, False, "edge")(
        y2.reshape(NC * N, OUT), src3e, dst3e, zers)
    out = _dense3(agg2, cnt, hr, b_l2.reshape(1, OUT),
                  g_out.reshape(1, OUT), be_out.reshape(1, OUT))
    return out


# speedup vs baseline: 1.0401x; 1.0143x over previous
import functools

import jax
import jax.numpy as jnp
from jax import lax
from jax.experimental import pallas as pl
from jax.experimental.pallas import tpu as pltpu
from jax.experimental.pallas import tpu_sc as plsc

N = 10000
E = 160000
F_IN = 256
HID = 256
OUT = 128

NC = 2
NS = 16
CH = 128
EP = 163840
NPAD = 16
CNTP = 10112

_PREC = jax.lax.Precision.DEFAULT


_RB = 2000


def _silu(v):
    return v * (1.0 / (1.0 + jnp.exp(-v)))


def _ln(v, g, b, eps=1e-5):
    mu = jnp.mean(v, axis=-1, keepdims=True)
    var = jnp.mean((v - mu) * (v - mu), axis=-1, keepdims=True)
    return (v - mu) / jnp.sqrt(var + eps) * g + b


def _dense1_body(x_ref, wl1_ref, wr1_ref, wlin_ref, blin_ref,
                 y1_ref, xr_ref, res_ref):
    xa = x_ref[...]
    y1 = jnp.dot(xa, wl1_ref[...], preferred_element_type=jnp.float32,
                 precision=_PREC)
    y1_ref[0] = y1[:, :HID // 2]
    y1_ref[1] = y1[:, HID // 2:]
    xr_ref[...] = jnp.dot(xa, wr1_ref[...], preferred_element_type=jnp.float32,
                          precision=_PREC)
    res_ref[...] = jnp.dot(xa, wlin_ref[...], preferred_element_type=jnp.float32,
                           precision=_PREC) + blin_ref[...]


def _dense1(x, W_l1, W_r1, W_lin, b_lin2d):
    grid = (N // _RB,)
    return pl.pallas_call(
        _dense1_body,
        grid=grid,
        in_specs=[
            pl.BlockSpec((_RB, F_IN), lambda i: (i, 0)),
            pl.BlockSpec((F_IN, HID), lambda i: (0, 0)),
            pl.BlockSpec((F_IN, HID), lambda i: (0, 0)),
            pl.BlockSpec((F_IN, HID), lambda i: (0, 0)),
            pl.BlockSpec((1, HID), lambda i: (0, 0)),
        ],
        out_specs=[
            pl.BlockSpec((NC, _RB, HID // NC), lambda i: (0, i, 0)),
            pl.BlockSpec((_RB, HID), lambda i: (i, 0)),
            pl.BlockSpec((_RB, HID), lambda i: (i, 0)),
        ],
        out_shape=[
            jax.ShapeDtypeStruct((NC, N, HID // NC), jnp.float32),
            jax.ShapeDtypeStruct((N, HID), jnp.float32),
            jax.ShapeDtypeStruct((N, HID), jnp.float32),
        ],
    )(x, W_l1, W_r1, W_lin, b_lin2d)


def _dense2_body(agg_ref, cnt_ref, xr_ref, res_ref, bl1_ref, g1_ref, be1_ref,
                 g2_ref, be2_ref, wl2_ref, wr2_ref, y2_ref, hr_ref):
    cm = jnp.maximum(cnt_ref[...], 1.0)
    mean = jnp.concatenate([agg_ref[0], agg_ref[1]], axis=-1) / cm
    h = mean + bl1_ref[...] + xr_ref[...]
    h = _silu(h)
    h = _ln(h, g1_ref[...], be1_ref[...])
    h = res_ref[...] + h
    h = _silu(h)
    h = _ln(h, g2_ref[...], be2_ref[...])
    y2 = jnp.dot(h, wl2_ref[...], preferred_element_type=jnp.float32,
                 precision=_PREC)
    y2_ref[0] = y2
    y2_ref[1] = y2
    hr_ref[...] = jnp.dot(h, wr2_ref[...], preferred_element_type=jnp.float32,
                          precision=_PREC)


def _dense2(agg1, cnt, xr, res, b_l1, g1, be1, g2, be2, W_l2, W_r2):
    grid = (N // _RB,)
    vec = lambda: pl.BlockSpec((1, HID), lambda i: (0, 0))
    return pl.pallas_call(
        _dense2_body,
        grid=grid,
        in_specs=[
            pl.BlockSpec((NC, _RB, HID // NC), lambda i: (0, i, 0)),
            pl.BlockSpec((_RB, 1), lambda i: (i, 0)),
            pl.BlockSpec((_RB, HID), lambda i: (i, 0)),
            pl.BlockSpec((_RB, HID), lambda i: (i, 0)),
            vec(), vec(), vec(), vec(), vec(),
            pl.BlockSpec((HID, OUT), lambda i: (0, 0)),
            pl.BlockSpec((HID, OUT), lambda i: (0, 0)),
        ],
        out_specs=[
            pl.BlockSpec((NC, _RB, OUT), lambda i: (0, i, 0)),
            pl.BlockSpec((_RB, OUT), lambda i: (i, 0)),
        ],
        out_shape=[
            jax.ShapeDtypeStruct((NC, N, OUT), jnp.float32),
            jax.ShapeDtypeStruct((N, OUT), jnp.float32),
        ],
    )(agg1, cnt, xr, res, b_l1, g1, be1, g2, be2, W_l2, W_r2)


def _dense3_body(agg_ref, cnt_ref, hr_ref, bl2_ref, go_ref, beo_ref, out_ref):
    cm = jnp.maximum(cnt_ref[...], 1.0)
    mean = (agg_ref[0] + agg_ref[1]) / cm
    h = mean + bl2_ref[...] + hr_ref[...]
    out_ref[...] = _ln(h, go_ref[...], beo_ref[...])


def _dense3(agg2, cnt, hr, b_l2, g_out, be_out):
    grid = (N // _RB,)
    vec = lambda: pl.BlockSpec((1, OUT), lambda i: (0, 0))
    return pl.pallas_call(
        _dense3_body,
        grid=grid,
        in_specs=[
            pl.BlockSpec((NC, _RB, OUT), lambda i: (0, i, 0)),
            pl.BlockSpec((_RB, 1), lambda i: (i, 0)),
            pl.BlockSpec((_RB, OUT), lambda i: (i, 0)),
            vec(), vec(), vec(),
        ],
        out_specs=pl.BlockSpec((_RB, OUT), lambda i: (i, 0)),
        out_shape=jax.ShapeDtypeStruct((N, OUT), jnp.float32),
    )(agg2, cnt, hr, b_l2, g_out, be_out)


def _make_seg(d, with_cnt, split):
    per_core = (EP // CH) // (1 if split == "feat" else NC)
    base = per_core // NS
    pblk = 40
    nph = base // pblk
    assert base == nph * pblk
    rq = (N // NS) // 8 * 8
    tail = N - rq * NS

    mesh = plsc.VectorSubcoreMesh(core_axis_name="c", subcore_axis_name="s",
                                  num_cores=NC, num_subcores=NS)

    out_type = [jax.ShapeDtypeStruct((NC, N, d), jnp.float32)]
    scratch = [
        pltpu.VMEM((pblk, CH), jnp.int32),
        pltpu.VMEM((pblk, CH), jnp.int32),
        pltpu.VMEM((CH, d), jnp.float32),
        pltpu.VMEM((CH, d), jnp.float32),
        pltpu.VMEM_SHARED((N + NPAD, d), jnp.float32),
        pltpu.SemaphoreType.DMA,
        pltpu.SemaphoreType.DMA,
        pltpu.SemaphoreType.DMA,
        pltpu.SemaphoreType.DMA,
        pltpu.SemaphoreType.DMA,
    ]
    if with_cnt:
        out_type.append(jax.ShapeDtypeStruct((NC, CNTP), jnp.float32))
        scratch += [
            pltpu.VMEM((5 * CH,), jnp.float32),
            pltpu.VMEM_SHARED((CNTP,), jnp.float32),
        ]
    cq = [5 * CH] * (NS - 1) + [CNTP - 5 * CH * (NS - 1)]
    assert cq[-1] > 0 and cq[-1] % CH == 0

    def body(tbl, src3, dst3, zer, out, *rest):
        if with_cnt:
            (cnt_out, sidx, didx, rbufa, rbufb, acc, sga, sgb, ssa, ssb, semc,
             ones, cacc) = rest
        else:
            sidx, didx, rbufa, rbufb, acc, sga, sgb, ssa, ssb, semc = rest
        c = lax.axis_index("c")
        s = lax.axis_index("s")
        start = s * base


        pltpu.sync_copy(zer.at[pl.ds(s * rq, rq)], acc.at[pl.ds(s * rq, rq)])
        @pl.when(s == NS - 1)
        def _():
            pltpu.sync_copy(zer.at[pl.ds(NS * rq, tail)],
                            acc.at[pl.ds(NS * rq, tail)])
        if with_cnt:
            def fill(i, _):
                ones[pl.ds(i * 16, 16)] = jnp.where(
                    i < CH // 16, 1.0, 0.0) * jnp.ones((16,), jnp.float32)
                return 0
            lax.fori_loop(0, 5 * CH // 16, fill, 0, unroll=False)
            @pl.when(s < NS - 1)
            def _():
                pltpu.sync_copy(ones.at[pl.ds(CH, 4 * CH)],
                                cacc.at[pl.ds(s * 5 * CH, 4 * CH)])
                pltpu.sync_copy(ones.at[pl.ds(CH, CH)],
                                cacc.at[pl.ds(s * 5 * CH + 4 * CH, CH)])
            @pl.when(s == NS - 1)
            def _():
                pltpu.sync_copy(ones.at[pl.ds(CH, 4 * CH)],
                                cacc.at[pl.ds((NS - 1) * 5 * CH, 4 * CH)])

        plsc.subcore_barrier()

        for ph in range(nph):
            pltpu.sync_copy(src3.at[c, pl.ds(start + ph * pblk, pblk)], sidx)
            pltpu.sync_copy(dst3.at[c, pl.ds(start + ph * pblk, pblk)], didx)
            pltpu.async_copy(tbl.at[sidx.at[0]], rbufa, sga)

            def chunk(j, _):
                def work(cur, oth, sg_c, ss_c, sg_o, ss_o):
                    @pl.when(j >= 1)
                    def _():
                        pltpu.make_async_copy(
                            oth, acc.at[didx.at[0]], ss_o).wait()
                    @pl.when(j < pblk - 1)
                    def _():
                        pltpu.async_copy(tbl.at[sidx.at[j + 1]], oth, sg_o)
                    pltpu.make_async_copy(tbl.at[sidx.at[j]], cur, sg_c).wait()
                    pltpu.async_copy(cur, acc.at[didx.at[j]], ss_c, add=True)
                @pl.when(lax.rem(j, 2) == 0)
                def _():
                    work(rbufa, rbufb, sga, ssa, sgb, ssb)
                @pl.when(lax.rem(j, 2) == 1)
                def _():
                    work(rbufb, rbufa, sgb, ssb, sga, ssa)
                if with_cnt:
                    @pl.when(c == ph % NC)
                    def _():
                        @pl.when(j >= 1)
                        def _():
                            pltpu.make_async_copy(
                                ones.at[pl.ds(0, CH)], cacc.at[didx.at[0]],
                                semc).wait()
                        pltpu.async_copy(ones.at[pl.ds(0, CH)],
                                         cacc.at[didx.at[j]], semc, add=True)
                return 0

            lax.fori_loop(0, pblk, chunk, 0, unroll=False)

            last = rbufb if (pblk - 1) % 2 else rbufa
            lsem = ssb if (pblk - 1) % 2 else ssa
            pltpu.make_async_copy(last, acc.at[didx.at[0]], lsem).wait()
            if with_cnt:
                @pl.when(c == ph % NC)
                def _():
                    pltpu.make_async_copy(ones.at[pl.ds(0, CH)],
                                          cacc.at[didx.at[0]], semc).wait()

        plsc.subcore_barrier()

        pltpu.sync_copy(acc.at[pl.ds(s * rq, rq)],
                        out.at[c, pl.ds(s * rq, rq)])
        @pl.when(s == NS - 1)
        def _():
            pltpu.sync_copy(acc.at[pl.ds(NS * rq, tail)],
                            out.at[c, pl.ds(NS * rq, tail)])
        if with_cnt:
            @pl.when(s < NS - 1)
            def _():
                pltpu.sync_copy(cacc.at[pl.ds(s * 5 * CH, 5 * CH)],
                                cnt_out.at[c, pl.ds(s * 5 * CH, 5 * CH)])
            @pl.when(s == NS - 1)
            def _():
                pltpu.sync_copy(cacc.at[pl.ds((NS - 1) * 5 * CH, 4 * CH)],
                                cnt_out.at[c, pl.ds((NS - 1) * 5 * CH, 4 * CH)])

    return pl.kernel(body, out_type=out_type, mesh=mesh, scratch_types=scratch)


def kernel(x, edge_index, W_lin, b_lin, W_l1, b_l1, W_r1, W_l2, b_l2, W_r2,
           g1, be1, g2, be2, g_out, be_out):
    src = edge_index[0].astype(jnp.int32)
    dst = edge_index[1].astype(jnp.int32)
    nch_all = EP // CH
    pad_per = CH - E // nch_all
    dsrc = jnp.zeros((nch_all, pad_per), jnp.int32)
    ddst = N + (jnp.arange(nch_all * pad_per, dtype=jnp.int32)
                % NPAD).reshape(nch_all, pad_per)
    src = jnp.concatenate([src.reshape(nch_all, CH - pad_per), dsrc], axis=1)
    dst = jnp.concatenate([dst.reshape(nch_all, CH - pad_per), ddst], axis=1)
    src = src.reshape(EP)
    dst = dst.reshape(EP)
    src3 = jnp.stack([src, src + N]).reshape(NC, EP // CH, CH)
    dst3 = jnp.tile(dst.reshape(1, EP // CH, CH), (NC, 1, 1))
    src3e = (src.reshape(NC, EP // CH // NC, CH)
             + (jnp.arange(NC, dtype=jnp.int32) * N)[:, None, None])
    dst3e = dst.reshape(NC, EP // CH // NC, CH)

    zers = jnp.zeros((N + NPAD, HID // NC), jnp.float32)

    y1t, xr, res = _dense1(x, W_l1, W_r1, W_lin, b_lin.reshape(1, HID))
    agg1, cnt2 = _make_seg(HID // NC, True, "feat")(
        y1t.reshape(NC * N, HID // NC), src3, dst3, zers)
    cnt = (cnt2[0] + cnt2[1])[:N].reshape(N, 1)
    y2, hr = _dense2(agg1, cnt, xr, res,
                     b_l1.reshape(1, HID), g1.reshape(1, HID),
                     be1.reshape(1, HID), g2.reshape(1, HID),
                     be2.reshape(1, HID), W_l2, W_r2)
    (agg2,) = _make_seg(OUT, False, "edge")(
        y2.reshape(NC * N, OUT), src3e, dst3e, zers)
    out = _dense3(agg2, cnt, hr, b_l2.reshape(1, OUT),
                  g_out.reshape(1, OUT), be_out.reshape(1, OUT))
    return out
